# baseline (speedup 1.0000x reference)
import functools

import jax
import jax.numpy as jnp
from jax import lax
from jax.experimental import pallas as pl
from jax.experimental.pallas import tpu as pltpu
from jax.experimental.pallas import tpu_sc as plsc

N_IN = 6890
N_OUT = 10475
BATCH = 1024

L = 16
NVP = 10480
NVEC = NVP // L
B_PER_TILE = BATCH // 32
BW = N_OUT * 3
ROW_W = 6704
RING = 6
RINGW = RING * ROW_W
ROWS_PER_TILE = B_PER_TILE * BW // ROW_W


def _spmm_sc(inp_vertices, cols3, vals_t):
    mesh = plsc.VectorSubcoreMesh(core_axis_name="c", subcore_axis_name="s")

    @functools.partial(
        pl.kernel,
        out_type=jax.ShapeDtypeStruct((BATCH * BW,), jnp.float32),
        mesh=mesh,
        compiler_params=pltpu.CompilerParams(
            use_tc_tiling_on_sc=False, needs_layout_passes=False),
        scratch_types=[
            pltpu.VMEM((3, NVP), jnp.int32),
            pltpu.VMEM((3, NVP), jnp.float32),
            pltpu.VMEM((N_IN * 3,), jnp.float32),
            pltpu.VMEM((RINGW,), jnp.float32),
            pltpu.SemaphoreType.DMA,
            pltpu.SemaphoreType.DMA,
        ],
    )
    def k(inp_hbm, cols_hbm, vals_hbm, out_hbm,
          cols_v, vals_v, inp_v, ring_v, in_s, out_s):
        t = lax.axis_index("s") * 2 + lax.axis_index("c")
        b0 = t * B_PER_TILE
        tile_base = t * (B_PER_TILE * BW)

        pltpu.sync_copy(cols_hbm, cols_v)
        pltpu.sync_copy(vals_hbm, vals_v)

        lane = lax.iota(jnp.int32, L)

        def in_copy(b):
            return pltpu.make_async_copy(inp_hbm.at[b], inp_v, in_s)

        def row_copy(rp, gr):
            return pltpu.make_async_copy(
                ring_v.at[pl.ds(rp * ROW_W, ROW_W)],
                out_hbm.at[pl.ds(tile_base + gr * ROW_W, ROW_W)],
                out_s)

        def compute(inp_ref, rb):
            def body(i, carry):
                o = i * L
                c0 = cols_v[0, pl.ds(o, L)]
                c1 = cols_v[1, pl.ds(o, L)]
                c2 = cols_v[2, pl.ds(o, L)]
                w0 = vals_v[0, pl.ds(o, L)]
                w1 = vals_v[1, pl.ds(o, L)]
                w2 = vals_v[2, pl.ds(o, L)]
                orow = lane + o
                m = orow < N_OUT
                fo = orow * 3 + rb
                for c in range(3):
                    g0 = plsc.load_gather(inp_ref, [c0 + c])
                    g1 = plsc.load_gather(inp_ref, [c1 + c])
                    g2 = plsc.load_gather(inp_ref, [c2 + c])
                    acc = g0 * w0 + g1 * w1 + g2 * w2
                    idx = fo + c
                    idx = jnp.where(idx >= RINGW, idx - RINGW, idx)
                    plsc.store_scatter(ring_v, [idx], acc, mask=m)
                return carry
            lax.fori_loop(0, NVEC, body, 0)

        def b_iter(j, carry):
            b = b0 + j
            copy = in_copy(b)
            copy.start()
            copy.wait()
            compute(inp_v, lax.rem(j * BW, RINGW))
            e0 = (75 * j) // 16
            n = (75 * (j + 1)) // 16 - e0
            for r in range(4):
                row_copy(lax.rem(e0 + r, RING), e0 + r).start()
            @pl.when(n > 4)
            def _():
                row_copy(lax.rem(e0 + 4, RING), e0 + 4).start()
            for r in range(4):
                row_copy(0, 0).wait()
            @pl.when(n > 4)
            def _():
                row_copy(0, 0).wait()
            return carry

        lax.fori_loop(0, B_PER_TILE, b_iter, 0)

    return k(inp_vertices, cols3, vals_t)


def kernel(inp_vertices, csr_vals, csr_rows, csr_cols):
    del csr_rows
    pad = NVP - N_OUT
    cols3 = (csr_cols.astype(jnp.int32) * 3).reshape(N_OUT, 3).T
    vals_t = csr_vals.astype(jnp.float32).reshape(N_OUT, 3).T
    cols3 = jnp.pad(cols3, ((0, 0), (0, pad)))
    vals_t = jnp.pad(vals_t, ((0, 0), (0, pad)))

    out = _spmm_sc(inp_vertices.reshape(BATCH, N_IN * 3), cols3, vals_t)
    return out.reshape(BATCH, N_OUT, 3)


# speedup vs baseline: 5.5001x; 5.5001x over previous
import functools

import jax
import jax.numpy as jnp
from jax import lax
from jax.experimental import pallas as pl
from jax.experimental.pallas import tpu as pltpu
from jax.experimental.pallas import tpu_sc as plsc

N_IN = 6890
N_OUT = 10475
BATCH = 1024

L = 16
GRP = 419
ROWS_PER_B = N_OUT // GRP
ROW_W = GRP * 3
ROW_P = 1264
GPH = 13
VPH = GPH * GRP
VOFF_STEP = (ROWS_PER_B - GPH) * GRP
GOFF_STEP = ROWS_PER_B - GPH
VBUF = 5456
NVEC = VBUF // L
B_PER_TILE = BATCH // 16


def _spmm_sc(inp_vertices, cols_p, vals_p):
    mesh = plsc.VectorSubcoreMesh(core_axis_name="c", subcore_axis_name="s")

    @functools.partial(
        pl.kernel,
        out_type=jax.ShapeDtypeStruct((BATCH * ROWS_PER_B, ROW_P), jnp.float32),
        mesh=mesh,
        compiler_params=pltpu.CompilerParams(
            use_tc_tiling_on_sc=False, needs_layout_passes=False),
        scratch_types=[
            pltpu.VMEM((3, VBUF), jnp.int32),
            pltpu.VMEM((3, VBUF), jnp.float32),
            pltpu.VMEM((N_IN * 3,), jnp.float32),
            pltpu.VMEM((N_IN * 3,), jnp.float32),
            pltpu.VMEM((GPH, ROW_P), jnp.float32),
            pltpu.VMEM((GPH, ROW_P), jnp.float32),
            pltpu.VMEM((GPH,), jnp.int32),
            pltpu.VMEM((GPH,), jnp.int32),
            pltpu.SemaphoreType.DMA,
            pltpu.SemaphoreType.DMA,
            pltpu.SemaphoreType.DMA,
            pltpu.SemaphoreType.DMA,
        ],
    )
    def k(inp_hbm, cols_hbm, vals_hbm, out_hbm,
          cols_v, vals_v, inp_v0, inp_v1, out_v0, out_v1, idx_v0, idx_v1,
          in_s0, in_s1, out_s0, out_s1):
        bg = lax.axis_index("s")
        vh = lax.axis_index("c")
        b0 = bg * B_PER_TILE

        pltpu.sync_copy(cols_hbm.at[vh], cols_v)
        pltpu.sync_copy(vals_hbm.at[vh], vals_v)

        inp_bufs = (inp_v0, inp_v1)
        out_bufs = (out_v0, out_v1)
        idx_bufs = (idx_v0, idx_v1)
        in_sems = (in_s0, in_s1)
        out_sems = (out_s0, out_s1)

        lane = lax.iota(jnp.int32, L)
        idx_mask = lane < GPH
        grp_base = lane + vh * GOFF_STEP

        def in_copy(b, u):
            return pltpu.make_async_copy(
                inp_hbm.at[b], inp_bufs[u], in_sems[u])

        def out_copy(u):
            return pltpu.make_async_copy(
                out_bufs[u], out_hbm.at[idx_bufs[u]], out_sems[u])

        def compute(inp_ref, out_ref):
            def body(i, carry):
                o = i * L
                c0 = cols_v[0, pl.ds(o, L)]
                c1 = cols_v[1, pl.ds(o, L)]
                c2 = cols_v[2, pl.ds(o, L)]
                w0 = vals_v[0, pl.ds(o, L)]
                w1 = vals_v[1, pl.ds(o, L)]
                w2 = vals_v[2, pl.ds(o, L)]
                orow = lane + o
                fo = orow * 3
                m = orow < VPH
                for c in range(3):
                    g0 = plsc.load_gather(inp_ref, [c0 + c])
                    g1 = plsc.load_gather(inp_ref, [c1 + c])
                    g2 = plsc.load_gather(inp_ref, [c2 + c])
                    acc = g0 * w0 + g1 * w1 + g2 * w2
                    f = fo + c
                    row = f // ROW_W
                    plsc.store_scatter(out_ref, [row, f - row * ROW_W], acc,
                                       mask=m)
                return carry
            lax.fori_loop(0, NVEC, body, 0)

        in_copy(b0, 0).start()

        def b_iter(j, carry):
            for u in range(2):
                idx = j * 2 + u
                b = b0 + idx
                in_copy(b, u).wait()
                @pl.when(idx + 1 < B_PER_TILE)
                def _():
                    in_copy(b + 1, u ^ 1).start()
                @pl.when(j > 0)
                def _():
                    out_copy(u).wait()
                plsc.store_scatter(idx_bufs[u], [lane],
                                   grp_base + b * ROWS_PER_B, mask=idx_mask)
                compute(inp_bufs[u], out_bufs[u])
                out_copy(u).start()
            return carry

        lax.fori_loop(0, B_PER_TILE // 2, b_iter, 0)

        for u in range(2):
            out_copy(u).wait()

    return k(inp_vertices, cols_p, vals_p)


def kernel(inp_vertices, csr_vals, csr_rows, csr_cols):
    del csr_rows
    cols3 = (csr_cols.astype(jnp.int32) * 3).reshape(N_OUT, 3).T
    vals_t = csr_vals.astype(jnp.float32).reshape(N_OUT, 3).T
    pad = VBUF - VPH

    def pack(a):
        h0 = a[:, :VPH]
        h1 = a[:, VOFF_STEP:]
        return jnp.stack([
            jnp.pad(h0, ((0, 0), (0, pad))),
            jnp.pad(h1, ((0, 0), (0, pad))),
        ])

    out = _spmm_sc(inp_vertices.reshape(BATCH, N_IN * 3),
                   pack(cols3), pack(vals_t))
    out = out.reshape(BATCH, ROWS_PER_B, ROW_P)[:, :, :ROW_W]
    return out.reshape(BATCH, N_OUT, 3)
